# baseline (speedup 1.0000x reference)
import functools

import jax
import jax.numpy as jnp
from jax import lax
from jax.experimental import pallas as pl
from jax.experimental.pallas import tpu as pltpu
from jax.experimental.pallas import tpu_sc as plsc

EMBED_D = 316
BATCH = 4096
HIST = 20
B_TOTAL = BATCH * HIST
NUM_CORES = 2
NUM_SUBCORES = 16
NW = NUM_CORES * NUM_SUBCORES
B_PER_W = B_TOTAL // NW
BCHUNK = 2
CHUNK = BCHUNK * HIST
N_CHUNKS = B_PER_W // CHUNK

_mesh = plsc.VectorSubcoreMesh(core_axis_name="c", subcore_axis_name="s")


@functools.partial(
    pl.kernel,
    mesh=_mesh,
    out_type=jax.ShapeDtypeStruct((BATCH, HIST, EMBED_D), jnp.float32),
    scratch_types=[
        pltpu.VMEM((B_PER_W,), jnp.int32),
        pltpu.SemaphoreType.DMA,
        pltpu.SemaphoreType.DMA,
    ],
    compiler_params=pltpu.CompilerParams(
        use_tc_tiling_on_sc=True, needs_layout_passes=False
    ),
)
def _gather_kernel(idx_hbm, table_hbm, out_hbm, idx_v, sem0, sem1):
    wid = lax.axis_index("s") * NUM_CORES + lax.axis_index("c")
    base = wid * B_PER_W
    bbase = wid * (B_PER_W // HIST)
    sems = (sem0, sem1)
    lanes = lax.iota(jnp.int32, 16)

    pltpu.sync_copy(idx_hbm.at[pl.ds(base, B_PER_W)], idx_v)

    def issue(c, slot):
        sem = sems[slot]

        def row(r, carry):
            pos = c * CHUNK + r
            a = jnp.max(plsc.load_gather(idx_v, [lanes * 0 + pos]))
            b = bbase + pos // HIST
            h = pos - (pos // HIST) * HIST
            pltpu.async_copy(table_hbm.at[pl.ds(a, 1)],
                             out_hbm.at[b, pl.ds(h, 1)], sem)
            return carry

        lax.fori_loop(0, CHUNK, row, 0)

    def drain(slot):
        def row(r, carry):
            pltpu.make_async_copy(table_hbm.at[pl.ds(0, 1)],
                                  out_hbm.at[0, pl.ds(0, 1)],
                                  sems[slot]).wait()
            return carry

        lax.fori_loop(0, CHUNK, row, 0)

    issue(0, 0)

    def pair(g, carry):
        c = 2 * g

        @pl.when(c + 1 < N_CHUNKS)
        def _():
            issue(c + 1, 1)

        drain(0)

        @pl.when(c + 2 < N_CHUNKS)
        def _():
            issue(c + 2, 0)

        @pl.when(c + 1 < N_CHUNKS)
        def _():
            drain(1)

        return carry

    lax.fori_loop(0, N_CHUNKS // 2, pair, 0)


def kernel(indices, table):
    flat_idx = indices.reshape(-1)
    return _gather_kernel(flat_idx, table)


# speedup vs baseline: 8.9145x; 8.9145x over previous
import functools

import jax
import jax.numpy as jnp
from jax import lax
from jax.experimental import pallas as pl
from jax.experimental.pallas import tpu as pltpu
from jax.experimental.pallas import tpu_sc as plsc

EMBED_D = 316
BATCH = 4096
HIST = 20
B_TOTAL = BATCH * HIST
NUM_CORES = 2
NUM_SUBCORES = 16
NW = NUM_CORES * NUM_SUBCORES
B_PER_W = B_TOTAL // NW
CHUNK = 40
N_CHUNKS = B_PER_W // CHUNK

_mesh = plsc.VectorSubcoreMesh(core_axis_name="c", subcore_axis_name="s")


@functools.partial(
    pl.kernel,
    mesh=_mesh,
    out_type=jax.ShapeDtypeStruct((B_TOTAL, EMBED_D), jnp.float32),
    scratch_types=[
        pltpu.VMEM((B_PER_W,), jnp.int32),
        pltpu.VMEM((CHUNK, EMBED_D), jnp.float32),
        pltpu.VMEM((CHUNK, EMBED_D), jnp.float32),
        pltpu.SemaphoreType.DMA,
        pltpu.SemaphoreType.DMA,
        pltpu.SemaphoreType.DMA,
        pltpu.SemaphoreType.DMA,
    ],
    compiler_params=pltpu.CompilerParams(
        use_tc_tiling_on_sc=True, needs_layout_passes=False
    ),
)
def _gather_kernel(idx_hbm, table_hbm, out_hbm, idx_v,
                   packed0_v, packed1_v, sem0, sem1, wsem0, wsem1):
    wid = lax.axis_index("s") * NUM_CORES + lax.axis_index("c")
    base = wid * B_PER_W
    packs = (packed0_v, packed1_v)
    sems = (sem0, sem1)
    wsems = (wsem0, wsem1)
    lanes = lax.iota(jnp.int32, 16)

    pltpu.sync_copy(idx_hbm.at[pl.ds(base, B_PER_W)], idx_v)

    def out_window(c):
        return out_hbm.at[pl.ds(base + c * CHUNK, CHUNK)]

    def issue(c, slot):
        packed = packs[slot]
        sem = sems[slot]

        def row(r, carry):
            a = jnp.max(plsc.load_gather(idx_v, [lanes * 0 + c * CHUNK + r]))
            pltpu.async_copy(table_hbm.at[pl.ds(a, 1)],
                             packed.at[pl.ds(r, 1)], sem)
            return carry

        lax.fori_loop(0, CHUNK, row, 0)

    def process(c, slot):
        packed = packs[slot]
        pltpu.make_async_copy(table_hbm.at[pl.ds(0, CHUNK)], packed,
                              sems[slot]).wait()

        @pl.when(c >= 2)
        def _():
            pltpu.make_async_copy(packed, out_window(c), wsems[slot]).wait()

        pltpu.async_copy(packed, out_window(c), wsems[slot])

    issue(0, 0)

    def pair(g, carry):
        c = 2 * g

        @pl.when(c + 1 < N_CHUNKS)
        def _():
            issue(c + 1, 1)

        process(c, 0)

        @pl.when(c + 2 < N_CHUNKS)
        def _():
            issue(c + 2, 0)

        @pl.when(c + 1 < N_CHUNKS)
        def _():
            process(c + 1, 1)

        return carry

    lax.fori_loop(0, N_CHUNKS // 2, pair, 0)

    pltpu.make_async_copy(packed0_v, out_window(N_CHUNKS - 2), wsem0).wait()
    pltpu.make_async_copy(packed1_v, out_window(N_CHUNKS - 1), wsem1).wait()


def kernel(indices, table):
    flat_idx = indices.reshape(-1)
    out = _gather_kernel(flat_idx, table)
    return out.reshape(indices.shape + (table.shape[1],))
